# baseline (speedup 1.0000x reference)
import jax
import jax.numpy as jnp
from jax.experimental import pallas as pl

D = 768
NE = 8
ED = 192
R = 512


def _moe_tile(t_ref, wr_ref, wab_ref, w3_ref, g_ref, b_ref, o_ref):
    tb = t_ref[...]
    logits = jax.lax.dot_general(
        tb, wr_ref[...], (((1,), (1,)), ((), ())),
        preferred_element_type=jnp.float32)
    ids = jax.lax.broadcasted_iota(jnp.int32, (R, NE), 1)
    m0 = jnp.max(logits, axis=1, keepdims=True)
    i0 = jnp.min(jnp.where(logits == m0, ids, NE), axis=1, keepdims=True)
    masked = jnp.where(ids == i0, jnp.float32(-1e30), logits)
    m1 = jnp.max(masked, axis=1, keepdims=True)
    i1 = jnp.min(jnp.where(masked == m1, ids, NE), axis=1, keepdims=True)
    e1 = jnp.exp(m1 - m0)
    w0s = 1.0 / (1.0 + e1)
    w1s = e1 * w0s
    wmat = jnp.where(ids == i0, w0s, 0.0) + jnp.where(ids == i1, w1s, 0.0)

    ab = jax.lax.dot_general(
        tb, wab_ref[...], (((1,), (0,)), ((), ())),
        preferred_element_type=jnp.float32)
    a = ab[:, :NE * ED]
    bv = ab[:, NE * ED:]
    h = a * jax.nn.sigmoid(a) * bv
    parts = [h[:, ei * ED:(ei + 1) * ED] * wmat[:, ei:ei + 1]
             for ei in range(NE)]
    hw = jnp.concatenate(parts, axis=1).astype(jnp.bfloat16)
    acc = jax.lax.dot_general(
        hw, w3_ref[...], (((1,), (0,)), ((), ())),
        preferred_element_type=jnp.float32)

    mean = jnp.mean(acc, axis=1, keepdims=True)
    cent = acc - mean
    var = jnp.mean(cent * cent, axis=1, keepdims=True)
    o_ref[...] = (cent * jax.lax.rsqrt(var + 1e-5) * g_ref[...]
                  + b_ref[...]).astype(jnp.bfloat16)


def kernel(x, Wr, W0, W2, W3, gamma, beta):
    B, C, H, W = x.shape
    T = B * H * W
    tokens = jnp.transpose(x, (0, 2, 3, 1)).reshape(T, C).astype(jnp.bfloat16)
    wab = jnp.concatenate(
        [W0.reshape(NE * ED, D), W2.reshape(NE * ED, D)], axis=0
    ).T.astype(jnp.bfloat16)
    w3r = W3.transpose(0, 2, 1).reshape(NE * ED, D).astype(jnp.bfloat16)
    ln = pl.pallas_call(
        _moe_tile,
        grid=(T // R,),
        in_specs=[
            pl.BlockSpec((R, D), lambda i: (i, 0)),
            pl.BlockSpec((NE, D), lambda i: (0, 0)),
            pl.BlockSpec((D, 2 * NE * ED), lambda i: (0, 0)),
            pl.BlockSpec((NE * ED, D), lambda i: (0, 0)),
            pl.BlockSpec((1, D), lambda i: (0, 0)),
            pl.BlockSpec((1, D), lambda i: (0, 0)),
        ],
        out_specs=pl.BlockSpec((R, D), lambda i: (i, 0)),
        out_shape=jax.ShapeDtypeStruct((T, D), jnp.bfloat16),
    )(tokens, Wr.astype(jnp.bfloat16), wab, w3r,
      gamma.reshape(1, D), beta.reshape(1, D))
    return jnp.transpose(ln.reshape(B, H, W, C), (0, 3, 1, 2)) + x


# speedup vs baseline: 1.5958x; 1.5958x over previous
import jax
import jax.numpy as jnp
from jax.experimental import pallas as pl

D = 768
NE = 8
ED = 192
R = 512


def _moe_tile(t_ref, wr_ref, wab_ref, w3_ref, g_ref, b_ref, o_ref):
    t = t_ref[...]
    tb = t.astype(jnp.bfloat16)
    logits = jax.lax.dot_general(
        tb, wr_ref[...], (((1,), (1,)), ((), ())),
        preferred_element_type=jnp.float32)
    ids = jax.lax.broadcasted_iota(jnp.int32, (R, NE), 1)
    m0 = jnp.max(logits, axis=1, keepdims=True)
    i0 = jnp.min(jnp.where(logits == m0, ids, NE), axis=1, keepdims=True)
    masked = jnp.where(ids == i0, jnp.float32(-1e30), logits)
    m1 = jnp.max(masked, axis=1, keepdims=True)
    i1 = jnp.min(jnp.where(masked == m1, ids, NE), axis=1, keepdims=True)
    e1 = jnp.exp(m1 - m0)
    w0s = 1.0 / (1.0 + e1)
    w1s = e1 * w0s
    wmat = jnp.where(ids == i0, w0s, 0.0) + jnp.where(ids == i1, w1s, 0.0)

    ab = jax.lax.dot_general(
        tb, wab_ref[...], (((1,), (0,)), ((), ())),
        preferred_element_type=jnp.float32)
    a = ab[:, :NE * ED]
    bv = ab[:, NE * ED:]
    h = a * (0.5 + 0.5 * jnp.tanh(0.5 * a)) * bv
    parts = [h[:, ei * ED:(ei + 1) * ED] * wmat[:, ei:ei + 1]
             for ei in range(NE)]
    hw = jnp.concatenate(parts, axis=1).astype(jnp.bfloat16)
    acc = jax.lax.dot_general(
        hw, w3_ref[...], (((1,), (0,)), ((), ())),
        preferred_element_type=jnp.float32)

    mean = jnp.mean(acc, axis=1, keepdims=True)
    cent = acc - mean
    var = jnp.mean(cent * cent, axis=1, keepdims=True)
    o_ref[...] = cent * jax.lax.rsqrt(var + 1e-5) * g_ref[...] + b_ref[...] + t


def kernel(x, Wr, W0, W2, W3, gamma, beta):
    B, C, H, W = x.shape
    T = B * H * W
    tokens = jnp.transpose(x, (0, 2, 3, 1)).reshape(T, C)
    wab = jnp.concatenate(
        [W0.reshape(NE * ED, D), W2.reshape(NE * ED, D)], axis=0
    ).T.astype(jnp.bfloat16)
    w3r = W3.transpose(0, 2, 1).reshape(NE * ED, D).astype(jnp.bfloat16)
    out = pl.pallas_call(
        _moe_tile,
        grid=(T // R,),
        in_specs=[
            pl.BlockSpec((R, D), lambda i: (i, 0)),
            pl.BlockSpec((NE, D), lambda i: (0, 0)),
            pl.BlockSpec((D, 2 * NE * ED), lambda i: (0, 0)),
            pl.BlockSpec((NE * ED, D), lambda i: (0, 0)),
            pl.BlockSpec((1, D), lambda i: (0, 0)),
            pl.BlockSpec((1, D), lambda i: (0, 0)),
        ],
        out_specs=pl.BlockSpec((R, D), lambda i: (i, 0)),
        out_shape=jax.ShapeDtypeStruct((T, D), jnp.float32),
    )(tokens, Wr.astype(jnp.bfloat16), wab, w3r,
      gamma.reshape(1, D), beta.reshape(1, D))
    return jnp.transpose(out.reshape(B, H, W, C), (0, 3, 1, 2))


# speedup vs baseline: 1.6188x; 1.0144x over previous
import jax
import jax.numpy as jnp
from jax.experimental import pallas as pl

D = 768
NE = 8
ED = 192
R = 512


def _moe_tile(t_ref, wr_ref, wab_ref, w3_ref, g_ref, b_ref, o_ref):
    t = t_ref[...]
    tb = t.astype(jnp.bfloat16)
    logits = jax.lax.dot_general(
        tb, wr_ref[...], (((1,), (1,)), ((), ())),
        preferred_element_type=jnp.float32)
    ids = jax.lax.broadcasted_iota(jnp.int32, (R, NE), 1)
    m0 = jnp.max(logits, axis=1, keepdims=True)
    i0 = jnp.min(jnp.where(logits == m0, ids, NE), axis=1, keepdims=True)
    masked = jnp.where(ids == i0, jnp.float32(-1e30), logits)
    m1 = jnp.max(masked, axis=1, keepdims=True)
    i1 = jnp.min(jnp.where(masked == m1, ids, NE), axis=1, keepdims=True)
    e1 = jnp.exp(m1 - m0)
    w0s = 1.0 / (1.0 + e1)
    w1s = e1 * w0s
    wmat = jnp.where(ids == i0, w0s, 0.0) + jnp.where(ids == i1, w1s, 0.0)

    ab = jax.lax.dot_general(
        tb, wab_ref[...], (((1,), (0,)), ((), ())),
        preferred_element_type=jnp.float32)
    a = ab[:, :NE * ED]
    bv = ab[:, NE * ED:]
    h = a * (0.5 + 0.5 * jnp.tanh(0.5 * a)) * bv
    parts = [h[:, ei * ED:(ei + 1) * ED] * wmat[:, ei:ei + 1]
             for ei in range(NE)]
    hw = jnp.concatenate(parts, axis=1).astype(jnp.bfloat16)
    acc = jax.lax.dot_general(
        hw, w3_ref[...], (((1,), (0,)), ((), ())),
        preferred_element_type=jnp.float32)

    mean = jnp.mean(acc, axis=1, keepdims=True)
    cent = acc - mean
    var = jnp.mean(acc * acc, axis=1, keepdims=True) - mean * mean
    o_ref[...] = cent * jax.lax.rsqrt(var + 1e-5) * g_ref[...] + b_ref[...] + t


def kernel(x, Wr, W0, W2, W3, gamma, beta):
    B, C, H, W = x.shape
    T = B * H * W
    tokens = jnp.transpose(x, (0, 2, 3, 1)).reshape(T, C)
    wab = jnp.concatenate(
        [W0.reshape(NE * ED, D), W2.reshape(NE * ED, D)], axis=0
    ).T.astype(jnp.bfloat16)
    w3r = W3.transpose(0, 2, 1).reshape(NE * ED, D).astype(jnp.bfloat16)
    out = pl.pallas_call(
        _moe_tile,
        grid=(T // R,),
        in_specs=[
            pl.BlockSpec((R, D), lambda i: (i, 0)),
            pl.BlockSpec((NE, D), lambda i: (0, 0)),
            pl.BlockSpec((D, 2 * NE * ED), lambda i: (0, 0)),
            pl.BlockSpec((NE * ED, D), lambda i: (0, 0)),
            pl.BlockSpec((1, D), lambda i: (0, 0)),
            pl.BlockSpec((1, D), lambda i: (0, 0)),
        ],
        out_specs=pl.BlockSpec((R, D), lambda i: (i, 0)),
        out_shape=jax.ShapeDtypeStruct((T, D), jnp.float32),
    )(tokens, Wr.astype(jnp.bfloat16), wab, w3r,
      gamma.reshape(1, D), beta.reshape(1, D))
    return jnp.transpose(out.reshape(B, H, W, C), (0, 3, 1, 2))
